# baseline (speedup 1.0000x reference)
import functools

import jax
import jax.numpy as jnp
from jax import lax
from jax.experimental import pallas as pl
from jax.experimental.pallas import tpu as pltpu
from jax.experimental.pallas import tpu_sc as plsc

N = 10000
NP = 10240
D = 256
DS = 128
E = 160000
CH = 128
E2 = 163840
EROWS = E2 // CH
NCHUNK = EROWS // 32
NCH2 = NCHUNK // 2
RB = 256

_mesh = plsc.VectorSubcoreMesh(
    core_axis_name="c", subcore_axis_name="s", num_cores=2, num_subcores=16)


def _deg_body(row2_hbm, degp_hbm, idx_v, ones_v, zseg_v, dacc):
    c = lax.axis_index("c")
    s = lax.axis_index("s")
    w = c * 16 + s

    for k in range(CH // 16):
        ones_v[pl.ds(k * 16, 16)] = jnp.ones((16,), jnp.float32)

    def zinit(i, _):
        zseg_v[pl.ds(i * 16, 16)] = jnp.zeros((16,), jnp.float32)
        return 0
    lax.fori_loop(0, 40, zinit, 0)
    pltpu.sync_copy(zseg_v, dacc.at[pl.ds(s * 640, 640)])

    pltpu.sync_copy(row2_hbm.at[pl.ds(w * NCHUNK, NCHUNK)], idx_v)

    plsc.subcore_barrier()

    def scat(j, _):
        pltpu.sync_copy(ones_v, dacc.at[idx_v.at[j]], add=True)
        return 0
    lax.fori_loop(0, NCHUNK, scat, 0)

    plsc.subcore_barrier()
    pltpu.sync_copy(dacc.at[pl.ds(s * 640, 640)],
                    degp_hbm.at[pl.ds(c * NP + s * 640, 640)])


_deg_call = functools.partial(
    pl.kernel, _deg_body, mesh=_mesh,
    out_type=jax.ShapeDtypeStruct((2 * NP,), jnp.float32),
    scratch_types=[
        pltpu.VMEM((NCHUNK, CH), jnp.int32),
        pltpu.VMEM((CH,), jnp.float32),
        pltpu.VMEM((640,), jnp.float32),
        pltpu.VMEM_SHARED((NP,), jnp.float32),
    ])()


def _prop_body(y_hbm, rowf_hbm, col2_hbm, out_hbm,
               rows_v, sidx_v, buf_v, acc, sem0, sem1, sem2, sem3):
    c = lax.axis_index("c")
    s = lax.axis_index("s")
    w = c * 16 + s

    pltpu.sync_copy(rowf_hbm.at[pl.ds(w * NCHUNK * CH, NCHUNK * CH)], rows_v)
    pltpu.sync_copy(col2_hbm.at[pl.ds(w * NCHUNK, NCHUNK)], sidx_v)

    def start(j, slot, sem):
        pltpu.async_copy(
            y_hbm.at[rows_v.at[pl.ds(j * CH, CH)], pl.ds(0, DS)],
            buf_v.at[slot], sem)

    def start2(j, slot, sem, t):
        if t == 0:
            start(j, slot, sem)
        else:
            pltpu.async_copy(
                y_hbm.at[rows_v.at[pl.ds(j * CH, CH)], pl.ds(DS, DS)],
                buf_v.at[slot], sem)

    def wait(slot, sem):
        pltpu.make_async_copy(y_hbm.at[pl.ds(0, CH), pl.ds(0, DS)],
                              buf_v.at[slot], sem).wait()

    for t in (0, 1):
        def zfill(i, _):
            buf_v[0, i // 8, pl.ds((i % 8) * 16, 16)] = jnp.zeros(
                (16,), jnp.float32)
            return 0
        lax.fori_loop(0, (CH * DS) // 16, zfill, 0)

        def zacc(i, _):
            pltpu.sync_copy(buf_v.at[0],
                            acc.at[pl.ds(s * 640 + i * CH, CH)])
            return 0
        lax.fori_loop(0, 640 // CH, zacc, 0)
        plsc.subcore_barrier()

        start2(0, 0, sem0, t)
        start2(1, 1, sem1, t)

        def scat(j, slot, sem):
            pltpu.async_copy(buf_v.at[slot], acc.at[sidx_v.at[j]], sem,
                             add=True)

        def swait(j, slot, sem):
            pltpu.make_async_copy(buf_v.at[slot], acc.at[sidx_v.at[j]],
                                  sem).wait()

        def body(i, _):
            j0 = 2 * i
            wait(0, sem0)
            scat(j0, 0, sem2)
            wait(1, sem1)
            scat(j0 + 1, 1, sem3)

            @pl.when(i < NCH2 - 1)
            def _():
                swait(j0, 0, sem2)
                start2(j0 + 2, 0, sem0, t)
                swait(j0 + 1, 1, sem3)
                start2(j0 + 3, 1, sem1, t)
            return 0
        lax.fori_loop(0, NCH2, body, 0)
        swait(NCHUNK - 2, 0, sem2)
        swait(NCHUNK - 1, 1, sem3)

        plsc.subcore_barrier()
        pltpu.sync_copy(
            acc.at[pl.ds(s * 640, 640)],
            out_hbm.at[pl.ds(c * NP + s * 640, 640), pl.ds(t * DS, DS)])
        plsc.subcore_barrier()


_prop_call = functools.partial(
    pl.kernel, _prop_body, mesh=_mesh,
    out_type=jax.ShapeDtypeStruct((2 * NP, D), jnp.float32),
    scratch_types=[
        pltpu.VMEM((NCHUNK * CH,), jnp.int32),
        pltpu.VMEM((NCHUNK, CH), jnp.int32),
        pltpu.VMEM((2, CH, DS), jnp.float32),
        pltpu.VMEM_SHARED((NP, DS), jnp.float32),
        pltpu.SemaphoreType.DMA,
        pltpu.SemaphoreType.DMA,
        pltpu.SemaphoreType.DMA,
        pltpu.SemaphoreType.DMA,
    ])()


def _tca_body(d0, d1, x, h0, h1, yx, yh0, yh1):
    dinv = lax.rsqrt(d0[...] + d1[...] + 2.0)
    yx[...] = dinv * x[...]
    yh0[...] = dinv * h0[...]
    yh1[...] = dinv * h1[...]


def _tca(d0, d1, x, h0, h1):
    col = pl.BlockSpec((RB, 1), lambda i: (i, 0))
    mat = pl.BlockSpec((RB, D), lambda i: (i, 0))
    return pl.pallas_call(
        _tca_body,
        grid=(NP // RB,),
        in_specs=[col, col, mat, mat, mat],
        out_specs=[mat, mat, mat],
        out_shape=[jax.ShapeDtypeStruct((NP, D), jnp.float32)] * 3,
    )(d0, d1, x, h0, h1)


_M0 = pl.BlockSpec((RB, D), lambda i: (i, 0))
_M1 = pl.BlockSpec((RB, D), lambda i: (i + NP // RB, 0))


def _tcb_body(sx0, sx1, sh0, sh1, yx, yh, hi, d0, d1, ws, bs, z, t1, q):
    dinv = lax.rsqrt(d0[...] + d1[...] + 2.0)
    ax = dinv * (sx0[...] + sx1[...] + 2.0 * yx[...])
    ah = dinv * (sh0[...] + sh1[...] + 2.0 * yh[...])
    axh = jnp.concatenate([ax, ah], axis=1)
    g = lax.dot_general(axh, ws[...], (((1,), (0,)), ((), ())),
                        precision=lax.Precision.DEFAULT,
                        preferred_element_type=jnp.float32) + bs[...]
    z[...] = jax.nn.sigmoid(g[:, :D])
    r = jax.nn.sigmoid(g[:, D:2 * D])
    t1[...] = g[:, 2 * D:]
    q[...] = dinv * (r * hi[...])


def _tcb(sx, sh, yx, yh, hi, d0, d1, ws, bs):
    col = pl.BlockSpec((RB, 1), lambda i: (i, 0))
    mat = pl.BlockSpec((RB, D), lambda i: (i, 0))
    return pl.pallas_call(
        _tcb_body,
        grid=(NP // RB,),
        in_specs=[_M0, _M1, _M0, _M1, mat, mat, mat, col, col,
                  pl.BlockSpec((2 * D, 3 * D), lambda i: (0, 0)),
                  pl.BlockSpec((1, 3 * D), lambda i: (0, 0))],
        out_specs=[mat, mat, mat],
        out_shape=[jax.ShapeDtypeStruct((NP, D), jnp.float32)] * 3,
    )(sx, sx, sh, sh, yx, yh, hi, d0, d1, ws, bs)


def _tcc_body(s20, s21, q, t1, z, hi, d0, d1, w5, b5, hn, yn):
    dinv = lax.rsqrt(d0[...] + d1[...] + 2.0)
    arh = dinv * (s20[...] + s21[...] + 2.0 * q[...])
    ht = jnp.tanh(t1[...] + b5[...] +
                  lax.dot_general(arh, w5[...], (((1,), (0,)), ((), ())),
                                  precision=lax.Precision.DEFAULT,
                                  preferred_element_type=jnp.float32))
    hv = z[...] * hi[...] + (1.0 - z[...]) * ht
    hn[...] = hv
    yn[...] = dinv * hv


def _tcc(s2, q, t1, z, hi, d0, d1, w5, b5):
    col = pl.BlockSpec((RB, 1), lambda i: (i, 0))
    mat = pl.BlockSpec((RB, D), lambda i: (i, 0))
    return pl.pallas_call(
        _tcc_body,
        grid=(NP // RB,),
        in_specs=[_M0, _M1, mat, mat, mat, mat, col, col,
                  pl.BlockSpec((D, D), lambda i: (0, 0)),
                  pl.BlockSpec((1, D), lambda i: (0, 0))],
        out_specs=[mat, mat],
        out_shape=[jax.ShapeDtypeStruct((NP, D), jnp.float32)] * 2,
    )(s2, s2, q, t1, z, hi, d0, d1, w5, b5)


def kernel(inp, edgidx, h, W, b):
    epad = N + (jnp.arange(E2 - E, dtype=jnp.int32) % (NP - N))
    rowf = jnp.concatenate([edgidx[0].astype(jnp.int32), epad])
    row2 = rowf.reshape(EROWS, CH)
    col2 = jnp.concatenate([edgidx[1].astype(jnp.int32), epad]).reshape(EROWS, CH)
    pad = jnp.zeros((NP - N, D), jnp.float32)
    inp_p = jnp.concatenate([inp, pad], axis=0)
    h0_p = jnp.concatenate([h[0], pad], axis=0)
    h1_p = jnp.concatenate([h[1], pad], axis=0)

    degp = _deg_call(row2)
    d0 = degp[:NP].reshape(NP, 1)
    d1 = degp[NP:].reshape(NP, 1)

    yx, yh0, yh1 = _tca(d0, d1, inp_p, h0_p, h1_p)

    hs = [None, None]
    yin = yx
    for i in range(2):
        hp = h0_p if i == 0 else h1_p
        yh = yh0 if i == 0 else yh1
        ws = jnp.concatenate([
            jnp.concatenate([W[i, 0], W[i, 1], W[i, 2]], axis=1),
            jnp.concatenate([W[i, 3], W[i, 4], jnp.zeros((D, D), jnp.float32)],
                            axis=1)], axis=0)
        bs = jnp.concatenate([b[i, 0] + b[i, 3], b[i, 1] + b[i, 4],
                              b[i, 2]])[None, :]
        sx = _prop_call(yin, rowf, col2)
        sh = _prop_call(yh, rowf, col2)
        z, t1, q = _tcb(sx, sh, yin, yh, hp, d0, d1, ws, bs)
        s2 = _prop_call(q, rowf, col2)
        hn, yn = _tcc(s2, q, t1, z, hp, d0, d1, W[i, 5], b[i, 5][None, :])
        hs[i] = hn[:N]
        yin = yn

    h_out = jnp.stack(hs, axis=0)
    return (h_out, h_out)


# speedup vs baseline: 1.0788x; 1.0788x over previous
import functools

import jax
import jax.numpy as jnp
from jax import lax
from jax.experimental import pallas as pl
from jax.experimental.pallas import tpu as pltpu
from jax.experimental.pallas import tpu_sc as plsc

N = 10000
NP = 10240
D = 256
DS = 128
E = 160000
CH = 128
E2 = 163840
EROWS = E2 // CH
NCHUNK = EROWS // 32
NCH2 = NCHUNK // 2
RB = 256

_mesh = plsc.VectorSubcoreMesh(
    core_axis_name="c", subcore_axis_name="s", num_cores=2, num_subcores=16)


def _deg_body(row2_hbm, degp_hbm, idx_v, ones_v, zseg_v, dacc):
    c = lax.axis_index("c")
    s = lax.axis_index("s")
    w = c * 16 + s

    for k in range(CH // 16):
        ones_v[pl.ds(k * 16, 16)] = jnp.ones((16,), jnp.float32)

    def zinit(i, _):
        zseg_v[pl.ds(i * 16, 16)] = jnp.zeros((16,), jnp.float32)
        return 0
    lax.fori_loop(0, 40, zinit, 0)
    pltpu.sync_copy(zseg_v, dacc.at[pl.ds(s * 640, 640)])

    pltpu.sync_copy(row2_hbm.at[pl.ds(w * NCHUNK, NCHUNK)], idx_v)

    plsc.subcore_barrier()

    def scat(j, _):
        pltpu.sync_copy(ones_v, dacc.at[idx_v.at[j]], add=True)
        return 0
    lax.fori_loop(0, NCHUNK, scat, 0)

    plsc.subcore_barrier()
    pltpu.sync_copy(dacc.at[pl.ds(s * 640, 640)],
                    degp_hbm.at[pl.ds(c * NP + s * 640, 640)])


_deg_call = functools.partial(
    pl.kernel, _deg_body, mesh=_mesh,
    out_type=jax.ShapeDtypeStruct((2 * NP,), jnp.float32),
    scratch_types=[
        pltpu.VMEM((NCHUNK, CH), jnp.int32),
        pltpu.VMEM((CH,), jnp.float32),
        pltpu.VMEM((640,), jnp.float32),
        pltpu.VMEM_SHARED((NP,), jnp.float32),
    ])()


def _prop_common(y_hbm, out_hbm, rows_v, sidx_v, buf_v, acc, sem0, sem1, c, s):
    def start2(j, slot, sem, t):
        pltpu.async_copy(
            y_hbm.at[rows_v.at[pl.ds(j * CH, CH)], pl.ds(t * DS, DS)],
            buf_v.at[slot], sem)

    def wait(slot, sem):
        pltpu.make_async_copy(y_hbm.at[pl.ds(0, CH), pl.ds(0, DS)],
                              buf_v.at[slot], sem).wait()

    for t in (0, 1):
        def zfill(i, _):
            buf_v[0, i // 8, pl.ds((i % 8) * 16, 16)] = jnp.zeros(
                (16,), jnp.float32)
            return 0
        lax.fori_loop(0, (CH * DS) // 16, zfill, 0)

        def zacc(i, _):
            pltpu.sync_copy(buf_v.at[0],
                            acc.at[pl.ds(s * 640 + i * CH, CH)])
            return 0
        lax.fori_loop(0, 640 // CH, zacc, 0)
        plsc.subcore_barrier()

        start2(0, 0, sem0, t)

        def body(i, _):
            j0 = 2 * i
            start2(j0 + 1, 1, sem1, t)
            wait(0, sem0)
            pltpu.sync_copy(buf_v.at[0], acc.at[sidx_v.at[j0]], add=True)

            @pl.when(i < NCH2 - 1)
            def _():
                start2(j0 + 2, 0, sem0, t)
            wait(1, sem1)
            pltpu.sync_copy(buf_v.at[1], acc.at[sidx_v.at[j0 + 1]], add=True)
            return 0
        lax.fori_loop(0, NCH2, body, 0)

        plsc.subcore_barrier()
        pltpu.sync_copy(
            acc.at[pl.ds(s * 640, 640)],
            out_hbm.at[pl.ds(c * NP + s * 640, 640), pl.ds(t * DS, DS)])
        plsc.subcore_barrier()


def _stage_edges(rowf_hbm, col2_hbm, rows_v, sidx_v, w):
    pltpu.sync_copy(rowf_hbm.at[pl.ds(w * NCHUNK * CH, NCHUNK * CH)], rows_v)
    pltpu.sync_copy(col2_hbm.at[pl.ds(w * NCHUNK, NCHUNK)], sidx_v)


def _prop_body(y_hbm, rowf_hbm, col2_hbm, out_hbm,
               rows_v, sidx_v, buf_v, acc, sem0, sem1):
    c = lax.axis_index("c")
    s = lax.axis_index("s")
    _stage_edges(rowf_hbm, col2_hbm, rows_v, sidx_v, c * 16 + s)
    _prop_common(y_hbm, out_hbm, rows_v, sidx_v, buf_v, acc, sem0, sem1, c, s)


def _prop2_body(ya_hbm, yb_hbm, rowf_hbm, col2_hbm, outa_hbm, outb_hbm,
                rows_v, sidx_v, buf_v, acc, sem0, sem1):
    c = lax.axis_index("c")
    s = lax.axis_index("s")
    _stage_edges(rowf_hbm, col2_hbm, rows_v, sidx_v, c * 16 + s)
    _prop_common(ya_hbm, outa_hbm, rows_v, sidx_v, buf_v, acc, sem0, sem1, c, s)
    _prop_common(yb_hbm, outb_hbm, rows_v, sidx_v, buf_v, acc, sem0, sem1, c, s)


_scratch = [
    pltpu.VMEM((NCHUNK * CH,), jnp.int32),
    pltpu.VMEM((NCHUNK, CH), jnp.int32),
    pltpu.VMEM((2, CH, DS), jnp.float32),
    pltpu.VMEM_SHARED((NP, DS), jnp.float32),
    pltpu.SemaphoreType.DMA,
    pltpu.SemaphoreType.DMA,
]

_prop2_call = functools.partial(
    pl.kernel, _prop2_body, mesh=_mesh,
    out_type=(jax.ShapeDtypeStruct((2 * NP, D), jnp.float32),
              jax.ShapeDtypeStruct((2 * NP, D), jnp.float32)),
    scratch_types=_scratch)()

_prop_call = functools.partial(
    pl.kernel, _prop_body, mesh=_mesh,
    out_type=jax.ShapeDtypeStruct((2 * NP, D), jnp.float32),
    scratch_types=_scratch)()


def _tca_body(d0, d1, x, h0, h1, yx, yh0, yh1):
    dinv = lax.rsqrt(d0[...] + d1[...] + 2.0)
    yx[...] = dinv * x[...]
    yh0[...] = dinv * h0[...]
    yh1[...] = dinv * h1[...]


def _tca(d0, d1, x, h0, h1):
    col = pl.BlockSpec((RB, 1), lambda i: (i, 0))
    mat = pl.BlockSpec((RB, D), lambda i: (i, 0))
    return pl.pallas_call(
        _tca_body,
        grid=(NP // RB,),
        in_specs=[col, col, mat, mat, mat],
        out_specs=[mat, mat, mat],
        out_shape=[jax.ShapeDtypeStruct((NP, D), jnp.float32)] * 3,
    )(d0, d1, x, h0, h1)


_M0 = pl.BlockSpec((RB, D), lambda i: (i, 0))
_M1 = pl.BlockSpec((RB, D), lambda i: (i + NP // RB, 0))


def _tcb_body(sx0, sx1, sh0, sh1, yx, yh, hi, d0, d1, ws, bs, z, t1, q):
    dinv = lax.rsqrt(d0[...] + d1[...] + 2.0)
    ax = dinv * (sx0[...] + sx1[...] + 2.0 * yx[...])
    ah = dinv * (sh0[...] + sh1[...] + 2.0 * yh[...])
    axh = jnp.concatenate([ax, ah], axis=1)
    g = lax.dot_general(axh, ws[...], (((1,), (0,)), ((), ())),
                        precision=lax.Precision.DEFAULT,
                        preferred_element_type=jnp.float32) + bs[...]
    z[...] = jax.nn.sigmoid(g[:, :D])
    r = jax.nn.sigmoid(g[:, D:2 * D])
    t1[...] = g[:, 2 * D:]
    q[...] = dinv * (r * hi[...])


def _tcb(sx, sh, yx, yh, hi, d0, d1, ws, bs):
    col = pl.BlockSpec((RB, 1), lambda i: (i, 0))
    mat = pl.BlockSpec((RB, D), lambda i: (i, 0))
    return pl.pallas_call(
        _tcb_body,
        grid=(NP // RB,),
        in_specs=[_M0, _M1, _M0, _M1, mat, mat, mat, col, col,
                  pl.BlockSpec((2 * D, 3 * D), lambda i: (0, 0)),
                  pl.BlockSpec((1, 3 * D), lambda i: (0, 0))],
        out_specs=[mat, mat, mat],
        out_shape=[jax.ShapeDtypeStruct((NP, D), jnp.float32)] * 3,
    )(sx, sx, sh, sh, yx, yh, hi, d0, d1, ws, bs)


def _tcc_body(s20, s21, q, t1, z, hi, d0, d1, w5, b5, hn, yn):
    dinv = lax.rsqrt(d0[...] + d1[...] + 2.0)
    arh = dinv * (s20[...] + s21[...] + 2.0 * q[...])
    ht = jnp.tanh(t1[...] + b5[...] +
                  lax.dot_general(arh, w5[...], (((1,), (0,)), ((), ())),
                                  precision=lax.Precision.DEFAULT,
                                  preferred_element_type=jnp.float32))
    hv = z[...] * hi[...] + (1.0 - z[...]) * ht
    hn[...] = hv
    yn[...] = dinv * hv


def _tcc(s2, q, t1, z, hi, d0, d1, w5, b5):
    col = pl.BlockSpec((RB, 1), lambda i: (i, 0))
    mat = pl.BlockSpec((RB, D), lambda i: (i, 0))
    return pl.pallas_call(
        _tcc_body,
        grid=(NP // RB,),
        in_specs=[_M0, _M1, mat, mat, mat, mat, col, col,
                  pl.BlockSpec((D, D), lambda i: (0, 0)),
                  pl.BlockSpec((1, D), lambda i: (0, 0))],
        out_specs=[mat, mat],
        out_shape=[jax.ShapeDtypeStruct((NP, D), jnp.float32)] * 2,
    )(s2, s2, q, t1, z, hi, d0, d1, w5, b5)


def kernel(inp, edgidx, h, W, b):
    epad = N + (jnp.arange(E2 - E, dtype=jnp.int32) % (NP - N))
    rowf = jnp.concatenate([edgidx[0].astype(jnp.int32), epad])
    row2 = rowf.reshape(EROWS, CH)
    col2 = jnp.concatenate([edgidx[1].astype(jnp.int32), epad]).reshape(EROWS, CH)
    pad = jnp.zeros((NP - N, D), jnp.float32)
    inp_p = jnp.concatenate([inp, pad], axis=0)
    h0_p = jnp.concatenate([h[0], pad], axis=0)
    h1_p = jnp.concatenate([h[1], pad], axis=0)

    degp = _deg_call(row2)
    d0 = degp[:NP].reshape(NP, 1)
    d1 = degp[NP:].reshape(NP, 1)

    yx, yh0, yh1 = _tca(d0, d1, inp_p, h0_p, h1_p)

    hs = [None, None]
    yin = yx
    for i in range(2):
        hp = h0_p if i == 0 else h1_p
        yh = yh0 if i == 0 else yh1
        ws = jnp.concatenate([
            jnp.concatenate([W[i, 0], W[i, 1], W[i, 2]], axis=1),
            jnp.concatenate([W[i, 3], W[i, 4], jnp.zeros((D, D), jnp.float32)],
                            axis=1)], axis=0)
        bs = jnp.concatenate([b[i, 0] + b[i, 3], b[i, 1] + b[i, 4],
                              b[i, 2]])[None, :]
        sx, sh = _prop2_call(yin, yh, rowf, col2)
        z, t1, q = _tcb(sx, sh, yin, yh, hp, d0, d1, ws, bs)
        s2 = _prop_call(q, rowf, col2)
        hn, yn = _tcc(s2, q, t1, z, hp, d0, d1, W[i, 5], b[i, 5][None, :])
        hs[i] = hn[:N]
        yin = yn

    h_out = jnp.stack(hs, axis=0)
    return (h_out, h_out)


# speedup vs baseline: 1.0963x; 1.0162x over previous
import functools

import jax
import jax.numpy as jnp
from jax import lax
from jax.experimental import pallas as pl
from jax.experimental.pallas import tpu as pltpu
from jax.experimental.pallas import tpu_sc as plsc

N = 10000
NP = 10240
D = 256
DS = 128
E = 160000
CH = 128
E2 = 163840
EROWS = E2 // CH
NCHUNK = EROWS // 32
NCH2 = NCHUNK // 2
RB = 256

_mesh = plsc.VectorSubcoreMesh(
    core_axis_name="c", subcore_axis_name="s", num_cores=2, num_subcores=16)


def _deg_body(row2_hbm, degp_hbm, idx_v, ones_v, zseg_v, dacc):
    c = lax.axis_index("c")
    s = lax.axis_index("s")
    w = c * 16 + s

    for k in range(CH // 16):
        ones_v[pl.ds(k * 16, 16)] = jnp.ones((16,), jnp.float32)

    def zinit(i, _):
        zseg_v[pl.ds(i * 16, 16)] = jnp.zeros((16,), jnp.float32)
        return 0
    lax.fori_loop(0, 40, zinit, 0)
    pltpu.sync_copy(zseg_v, dacc.at[pl.ds(s * 640, 640)])

    pltpu.sync_copy(row2_hbm.at[pl.ds(w * NCHUNK, NCHUNK)], idx_v)

    plsc.subcore_barrier()

    def scat(j, _):
        pltpu.sync_copy(ones_v, dacc.at[idx_v.at[j]], add=True)
        return 0
    lax.fori_loop(0, NCHUNK, scat, 0)

    plsc.subcore_barrier()
    pltpu.sync_copy(dacc.at[pl.ds(s * 640, 640)],
                    degp_hbm.at[pl.ds(c * NP + s * 640, 640)])


_deg_call = functools.partial(
    pl.kernel, _deg_body, mesh=_mesh,
    out_type=jax.ShapeDtypeStruct((2 * NP,), jnp.float32),
    scratch_types=[
        pltpu.VMEM((NCHUNK, CH), jnp.int32),
        pltpu.VMEM((CH,), jnp.float32),
        pltpu.VMEM((640,), jnp.float32),
        pltpu.VMEM_SHARED((NP,), jnp.float32),
    ])()


def _prop_common(y_hbm, out_hbm, rows_v, sidx_v, buf_v, acc, sem0, sem1, c, s):
    def start2(j, slot, sem, t):
        pltpu.async_copy(
            y_hbm.at[rows_v.at[pl.ds(j * CH, CH)], pl.ds(t * DS, DS)],
            buf_v.at[slot], sem)

    def wait(slot, sem):
        pltpu.make_async_copy(y_hbm.at[pl.ds(0, CH), pl.ds(0, DS)],
                              buf_v.at[slot], sem).wait()

    for t in (0, 1):
        def zfill(i, _):
            buf_v[0, i // 8, pl.ds((i % 8) * 16, 16)] = jnp.zeros(
                (16,), jnp.float32)
            return 0
        lax.fori_loop(0, (CH * DS) // 16, zfill, 0)

        def zacc(i, _):
            pltpu.sync_copy(buf_v.at[0],
                            acc.at[pl.ds(s * 640 + i * CH, CH)])
            return 0
        lax.fori_loop(0, 640 // CH, zacc, 0)
        plsc.subcore_barrier()

        start2(0, 0, sem0, t)

        def body(i, _):
            j0 = 2 * i
            start2(j0 + 1, 1, sem1, t)
            wait(0, sem0)
            pltpu.sync_copy(buf_v.at[0], acc.at[sidx_v.at[j0]], add=True)

            @pl.when(i < NCH2 - 1)
            def _():
                start2(j0 + 2, 0, sem0, t)
            wait(1, sem1)
            pltpu.sync_copy(buf_v.at[1], acc.at[sidx_v.at[j0 + 1]], add=True)
            return 0
        lax.fori_loop(0, NCH2, body, 0)

        plsc.subcore_barrier()
        pltpu.sync_copy(
            acc.at[pl.ds(s * 640, 640)],
            out_hbm.at[pl.ds(c * NP + s * 640, 640), pl.ds(t * DS, DS)])
        plsc.subcore_barrier()


def _stage_edges(rowf_hbm, col2_hbm, rows_v, sidx_v, w):
    pltpu.sync_copy(rowf_hbm.at[pl.ds(w * NCHUNK * CH, NCHUNK * CH)], rows_v)
    pltpu.sync_copy(col2_hbm.at[pl.ds(w * NCHUNK, NCHUNK)], sidx_v)


def _prop_body(y_hbm, rowf_hbm, col2_hbm, out_hbm,
               rows_v, sidx_v, buf_v, acc, sem0, sem1):
    c = lax.axis_index("c")
    s = lax.axis_index("s")
    _stage_edges(rowf_hbm, col2_hbm, rows_v, sidx_v, c * 16 + s)
    _prop_common(y_hbm, out_hbm, rows_v, sidx_v, buf_v, acc, sem0, sem1, c, s)


_scratch = [
    pltpu.VMEM((NCHUNK * CH,), jnp.int32),
    pltpu.VMEM((NCHUNK, CH), jnp.int32),
    pltpu.VMEM((2, CH, DS), jnp.float32),
    pltpu.VMEM_SHARED((NP, DS), jnp.float32),
    pltpu.SemaphoreType.DMA,
    pltpu.SemaphoreType.DMA,
]

_prop_call = functools.partial(
    pl.kernel, _prop_body, mesh=_mesh,
    out_type=jax.ShapeDtypeStruct((2 * NP, D), jnp.float32),
    scratch_types=_scratch)()


def _tca_body(d0, d1, x, h0, h1, yx, yh0, yh1):
    dinv = lax.rsqrt(d0[...] + d1[...] + 2.0)
    yx[...] = dinv * x[...]
    yh0[...] = dinv * h0[...]
    yh1[...] = dinv * h1[...]


def _tca(d0, d1, x, h0, h1):
    col = pl.BlockSpec((RB, 1), lambda i: (i, 0))
    mat = pl.BlockSpec((RB, D), lambda i: (i, 0))
    return pl.pallas_call(
        _tca_body,
        grid=(NP // RB,),
        in_specs=[col, col, mat, mat, mat],
        out_specs=[mat, mat, mat],
        out_shape=[jax.ShapeDtypeStruct((NP, D), jnp.float32)] * 3,
    )(d0, d1, x, h0, h1)


_M0 = pl.BlockSpec((RB, D), lambda i: (i, 0))
_M1 = pl.BlockSpec((RB, D), lambda i: (i + NP // RB, 0))


def _tcb_body(sx0, sx1, sh0, sh1, yx, yh, hi, d0, d1, ws, bs, z, t1, q):
    dinv = lax.rsqrt(d0[...] + d1[...] + 2.0)
    ax = dinv * (sx0[...] + sx1[...] + 2.0 * yx[...])
    ah = dinv * (sh0[...] + sh1[...] + 2.0 * yh[...])
    axh = jnp.concatenate([ax, ah], axis=1)
    g = lax.dot_general(axh, ws[...], (((1,), (0,)), ((), ())),
                        precision=lax.Precision.DEFAULT,
                        preferred_element_type=jnp.float32) + bs[...]
    z[...] = jax.nn.sigmoid(g[:, :D])
    r = jax.nn.sigmoid(g[:, D:2 * D])
    t1[...] = g[:, 2 * D:]
    q[...] = dinv * (r * hi[...])


def _tcb(sx, sh, yx, yh, hi, d0, d1, ws, bs):
    col = pl.BlockSpec((RB, 1), lambda i: (i, 0))
    mat = pl.BlockSpec((RB, D), lambda i: (i, 0))
    return pl.pallas_call(
        _tcb_body,
        grid=(NP // RB,),
        in_specs=[_M0, _M1, _M0, _M1, mat, mat, mat, col, col,
                  pl.BlockSpec((2 * D, 3 * D), lambda i: (0, 0)),
                  pl.BlockSpec((1, 3 * D), lambda i: (0, 0))],
        out_specs=[mat, mat, mat],
        out_shape=[jax.ShapeDtypeStruct((NP, D), jnp.float32)] * 3,
    )(sx, sx, sh, sh, yx, yh, hi, d0, d1, ws, bs)


def _tcc_body(s20, s21, q, t1, z, hi, d0, d1, w5, b5, hn, yn):
    dinv = lax.rsqrt(d0[...] + d1[...] + 2.0)
    arh = dinv * (s20[...] + s21[...] + 2.0 * q[...])
    ht = jnp.tanh(t1[...] + b5[...] +
                  lax.dot_general(arh, w5[...], (((1,), (0,)), ((), ())),
                                  precision=lax.Precision.DEFAULT,
                                  preferred_element_type=jnp.float32))
    hv = z[...] * hi[...] + (1.0 - z[...]) * ht
    hn[...] = hv
    yn[...] = dinv * hv


def _tcc(s2, q, t1, z, hi, d0, d1, w5, b5):
    col = pl.BlockSpec((RB, 1), lambda i: (i, 0))
    mat = pl.BlockSpec((RB, D), lambda i: (i, 0))
    return pl.pallas_call(
        _tcc_body,
        grid=(NP // RB,),
        in_specs=[_M0, _M1, mat, mat, mat, mat, col, col,
                  pl.BlockSpec((D, D), lambda i: (0, 0)),
                  pl.BlockSpec((1, D), lambda i: (0, 0))],
        out_specs=[mat, mat],
        out_shape=[jax.ShapeDtypeStruct((NP, D), jnp.float32)] * 2,
    )(s2, s2, q, t1, z, hi, d0, d1, w5, b5)


def kernel(inp, edgidx, h, W, b):
    epad = N + (jnp.arange(E2 - E, dtype=jnp.int32) % (NP - N))
    rowf = jnp.concatenate([edgidx[0].astype(jnp.int32), epad])
    row2 = rowf.reshape(EROWS, CH)
    col2 = jnp.concatenate([edgidx[1].astype(jnp.int32), epad]).reshape(EROWS, CH)
    pad = jnp.zeros((NP - N, D), jnp.float32)
    inp_p = jnp.concatenate([inp, pad], axis=0)
    h0_p = jnp.concatenate([h[0], pad], axis=0)
    h1_p = jnp.concatenate([h[1], pad], axis=0)

    degp = _deg_call(row2)
    d0 = degp[:NP].reshape(NP, 1)
    d1 = degp[NP:].reshape(NP, 1)

    yx, yh0, yh1 = _tca(d0, d1, inp_p, h0_p, h1_p)

    hs = [None, None]
    yin = yx
    for i in range(2):
        hp = h0_p if i == 0 else h1_p
        yh = yh0 if i == 0 else yh1
        ws = jnp.concatenate([
            jnp.concatenate([W[i, 0], W[i, 1], W[i, 2]], axis=1),
            jnp.concatenate([W[i, 3], W[i, 4], jnp.zeros((D, D), jnp.float32)],
                            axis=1)], axis=0)
        bs = jnp.concatenate([b[i, 0] + b[i, 3], b[i, 1] + b[i, 4],
                              b[i, 2]])[None, :]
        sx = _prop_call(yin, rowf, col2)
        sh = _prop_call(yh, rowf, col2)
        z, t1, q = _tcb(sx, sh, yin, yh, hp, d0, d1, ws, bs)
        s2 = _prop_call(q, rowf, col2)
        hn, yn = _tcc(s2, q, t1, z, hp, d0, d1, W[i, 5], b[i, 5][None, :])
        hs[i] = hn[:N]
        yin = yn

    h_out = jnp.stack(hs, axis=0)
    return (h_out, h_out)


# speedup vs baseline: 1.1468x; 1.0461x over previous
import functools

import jax
import jax.numpy as jnp
from jax import lax
from jax.experimental import pallas as pl
from jax.experimental.pallas import tpu as pltpu
from jax.experimental.pallas import tpu_sc as plsc

N = 10000
NP = 10240
D = 256
DS = 128
E = 160000
CH = 128
E2 = 163840
EROWS = E2 // CH
NCHUNK = EROWS // 32
NCH2 = NCHUNK // 2
RB = 512

_mesh = plsc.VectorSubcoreMesh(
    core_axis_name="c", subcore_axis_name="s", num_cores=2, num_subcores=16)


def _deg_body(row2_hbm, degp_hbm, idx_v, ones_v, zseg_v, dacc):
    c = lax.axis_index("c")
    s = lax.axis_index("s")
    w = c * 16 + s

    for k in range(CH // 16):
        ones_v[pl.ds(k * 16, 16)] = jnp.ones((16,), jnp.float32)

    def zinit(i, _):
        zseg_v[pl.ds(i * 16, 16)] = jnp.zeros((16,), jnp.float32)
        return 0
    lax.fori_loop(0, 40, zinit, 0)
    pltpu.sync_copy(zseg_v, dacc.at[pl.ds(s * 640, 640)])

    pltpu.sync_copy(row2_hbm.at[pl.ds(w * NCHUNK, NCHUNK)], idx_v)

    plsc.subcore_barrier()

    def scat(j, _):
        pltpu.sync_copy(ones_v, dacc.at[idx_v.at[j]], add=True)
        return 0
    lax.fori_loop(0, NCHUNK, scat, 0)

    plsc.subcore_barrier()
    pltpu.sync_copy(dacc.at[pl.ds(s * 640, 640)],
                    degp_hbm.at[pl.ds(c * NP + s * 640, 640)])


_deg_call = functools.partial(
    pl.kernel, _deg_body, mesh=_mesh,
    out_type=jax.ShapeDtypeStruct((2 * NP,), jnp.float32),
    scratch_types=[
        pltpu.VMEM((NCHUNK, CH), jnp.int32),
        pltpu.VMEM((CH,), jnp.float32),
        pltpu.VMEM((640,), jnp.float32),
        pltpu.VMEM_SHARED((NP,), jnp.float32),
    ])()


def _prop_common(y_hbm, out_hbm, rows_v, sidx_v, buf_v, acc, sem0, sem1, c, s):
    def start2(j, slot, sem, t):
        pltpu.async_copy(
            y_hbm.at[rows_v.at[pl.ds(j * CH, CH)], pl.ds(t * DS, DS)],
            buf_v.at[slot], sem)

    def wait(slot, sem):
        pltpu.make_async_copy(y_hbm.at[pl.ds(0, CH), pl.ds(0, DS)],
                              buf_v.at[slot], sem).wait()

    for t in (0, 1):
        def zfill(i, _):
            buf_v[0, i // 8, pl.ds((i % 8) * 16, 16)] = jnp.zeros(
                (16,), jnp.float32)
            return 0
        lax.fori_loop(0, (CH * DS) // 16, zfill, 0)

        def zacc(i, _):
            pltpu.sync_copy(buf_v.at[0],
                            acc.at[pl.ds(s * 640 + i * CH, CH)])
            return 0
        lax.fori_loop(0, 640 // CH, zacc, 0)
        plsc.subcore_barrier()

        start2(0, 0, sem0, t)

        def body(i, _):
            j0 = 2 * i
            start2(j0 + 1, 1, sem1, t)
            wait(0, sem0)
            pltpu.sync_copy(buf_v.at[0], acc.at[sidx_v.at[j0]], add=True)

            @pl.when(i < NCH2 - 1)
            def _():
                start2(j0 + 2, 0, sem0, t)
            wait(1, sem1)
            pltpu.sync_copy(buf_v.at[1], acc.at[sidx_v.at[j0 + 1]], add=True)
            return 0
        lax.fori_loop(0, NCH2, body, 0)

        plsc.subcore_barrier()
        pltpu.sync_copy(
            acc.at[pl.ds(s * 640, 640)],
            out_hbm.at[pl.ds(c * NP + s * 640, 640), pl.ds(t * DS, DS)])
        plsc.subcore_barrier()


def _stage_edges(rowf_hbm, col2_hbm, rows_v, sidx_v, w):
    pltpu.sync_copy(rowf_hbm.at[pl.ds(w * NCHUNK * CH, NCHUNK * CH)], rows_v)
    pltpu.sync_copy(col2_hbm.at[pl.ds(w * NCHUNK, NCHUNK)], sidx_v)


def _prop_body(y_hbm, rowf_hbm, col2_hbm, out_hbm,
               rows_v, sidx_v, buf_v, acc, sem0, sem1):
    c = lax.axis_index("c")
    s = lax.axis_index("s")
    _stage_edges(rowf_hbm, col2_hbm, rows_v, sidx_v, c * 16 + s)
    _prop_common(y_hbm, out_hbm, rows_v, sidx_v, buf_v, acc, sem0, sem1, c, s)


_scratch = [
    pltpu.VMEM((NCHUNK * CH,), jnp.int32),
    pltpu.VMEM((NCHUNK, CH), jnp.int32),
    pltpu.VMEM((2, CH, DS), jnp.float32),
    pltpu.VMEM_SHARED((NP, DS), jnp.float32),
    pltpu.SemaphoreType.DMA,
    pltpu.SemaphoreType.DMA,
]

_prop_call = functools.partial(
    pl.kernel, _prop_body, mesh=_mesh,
    out_type=jax.ShapeDtypeStruct((2 * NP, D), jnp.float32),
    scratch_types=_scratch)()


def _tca_body(d0, d1, x, h0, h1, yx, yh0, yh1):
    dinv = lax.rsqrt(d0[...] + d1[...] + 2.0)
    yx[...] = dinv * x[...]
    yh0[...] = dinv * h0[...]
    yh1[...] = dinv * h1[...]


def _tca(d0, d1, x, h0, h1):
    col = pl.BlockSpec((RB, 1), lambda i: (i, 0))
    mat = pl.BlockSpec((RB, D), lambda i: (i, 0))
    return pl.pallas_call(
        _tca_body,
        grid=(NP // RB,),
        in_specs=[col, col, mat, mat, mat],
        out_specs=[mat, mat, mat],
        out_shape=[jax.ShapeDtypeStruct((NP, D), jnp.float32)] * 3,
    )(d0, d1, x, h0, h1)


_M0 = pl.BlockSpec((RB, D), lambda i: (i, 0))
_M1 = pl.BlockSpec((RB, D), lambda i: (i + NP // RB, 0))


def _tcb_body(sx0, sx1, sh0, sh1, yx, yh, hi, d0, d1, ws, bs, z, t1, q):
    dinv = lax.rsqrt(d0[...] + d1[...] + 2.0)
    ax = dinv * (sx0[...] + sx1[...] + 2.0 * yx[...])
    ah = dinv * (sh0[...] + sh1[...] + 2.0 * yh[...])
    axh = jnp.concatenate([ax, ah], axis=1)
    g = lax.dot_general(axh, ws[...], (((1,), (0,)), ((), ())),
                        precision=lax.Precision.DEFAULT,
                        preferred_element_type=jnp.float32) + bs[...]
    z[...] = jax.nn.sigmoid(g[:, :D])
    r = jax.nn.sigmoid(g[:, D:2 * D])
    t1[...] = g[:, 2 * D:]
    q[...] = dinv * (r * hi[...])


def _tcb(sx, sh, yx, yh, hi, d0, d1, ws, bs):
    col = pl.BlockSpec((RB, 1), lambda i: (i, 0))
    mat = pl.BlockSpec((RB, D), lambda i: (i, 0))
    return pl.pallas_call(
        _tcb_body,
        grid=(NP // RB,),
        in_specs=[_M0, _M1, _M0, _M1, mat, mat, mat, col, col,
                  pl.BlockSpec((2 * D, 3 * D), lambda i: (0, 0)),
                  pl.BlockSpec((1, 3 * D), lambda i: (0, 0))],
        out_specs=[mat, mat, mat],
        out_shape=[jax.ShapeDtypeStruct((NP, D), jnp.float32)] * 3,
    )(sx, sx, sh, sh, yx, yh, hi, d0, d1, ws, bs)


def _tcc_body(s20, s21, q, t1, z, hi, d0, d1, w5, b5, hn, yn):
    dinv = lax.rsqrt(d0[...] + d1[...] + 2.0)
    arh = dinv * (s20[...] + s21[...] + 2.0 * q[...])
    ht = jnp.tanh(t1[...] + b5[...] +
                  lax.dot_general(arh, w5[...], (((1,), (0,)), ((), ())),
                                  precision=lax.Precision.DEFAULT,
                                  preferred_element_type=jnp.float32))
    hv = z[...] * hi[...] + (1.0 - z[...]) * ht
    hn[...] = hv
    yn[...] = dinv * hv


def _tcc(s2, q, t1, z, hi, d0, d1, w5, b5):
    col = pl.BlockSpec((RB, 1), lambda i: (i, 0))
    mat = pl.BlockSpec((RB, D), lambda i: (i, 0))
    return pl.pallas_call(
        _tcc_body,
        grid=(NP // RB,),
        in_specs=[_M0, _M1, mat, mat, mat, mat, col, col,
                  pl.BlockSpec((D, D), lambda i: (0, 0)),
                  pl.BlockSpec((1, D), lambda i: (0, 0))],
        out_specs=[mat, mat],
        out_shape=[jax.ShapeDtypeStruct((NP, D), jnp.float32)] * 2,
    )(s2, s2, q, t1, z, hi, d0, d1, w5, b5)


def kernel(inp, edgidx, h, W, b):
    epad = N + (jnp.arange(E2 - E, dtype=jnp.int32) % (NP - N))
    rowf = jnp.concatenate([edgidx[0].astype(jnp.int32), epad])
    row2 = rowf.reshape(EROWS, CH)
    col2 = jnp.concatenate([edgidx[1].astype(jnp.int32), epad]).reshape(EROWS, CH)
    pad = jnp.zeros((NP - N, D), jnp.float32)
    inp_p = jnp.concatenate([inp, pad], axis=0)
    h0_p = jnp.concatenate([h[0], pad], axis=0)
    h1_p = jnp.concatenate([h[1], pad], axis=0)

    degp = _deg_call(row2)
    d0 = degp[:NP].reshape(NP, 1)
    d1 = degp[NP:].reshape(NP, 1)

    yx, yh0, yh1 = _tca(d0, d1, inp_p, h0_p, h1_p)

    hs = [None, None]
    yin = yx
    for i in range(2):
        hp = h0_p if i == 0 else h1_p
        yh = yh0 if i == 0 else yh1
        ws = jnp.concatenate([
            jnp.concatenate([W[i, 0], W[i, 1], W[i, 2]], axis=1),
            jnp.concatenate([W[i, 3], W[i, 4], jnp.zeros((D, D), jnp.float32)],
                            axis=1)], axis=0)
        bs = jnp.concatenate([b[i, 0] + b[i, 3], b[i, 1] + b[i, 4],
                              b[i, 2]])[None, :]
        sx = _prop_call(yin, rowf, col2)
        sh = _prop_call(yh, rowf, col2)
        z, t1, q = _tcb(sx, sh, yin, yh, hp, d0, d1, ws, bs)
        s2 = _prop_call(q, rowf, col2)
        hn, yn = _tcc(s2, q, t1, z, hp, d0, d1, W[i, 5], b[i, 5][None, :])
        hs[i] = hn[:N]
        yin = yn

    h_out = jnp.stack(hs, axis=0)
    return (h_out, h_out)


# speedup vs baseline: 1.1582x; 1.0099x over previous
import functools

import jax
import jax.numpy as jnp
from jax import lax
from jax.experimental import pallas as pl
from jax.experimental.pallas import tpu as pltpu
from jax.experimental.pallas import tpu_sc as plsc

N = 10000
NP = 10240
D = 256
DS = 128
E = 160000
CH = 128
E2 = 163840
EROWS = E2 // CH
NCHUNK = EROWS // 32
NCH2 = NCHUNK // 2
RB = 1024

_mesh = plsc.VectorSubcoreMesh(
    core_axis_name="c", subcore_axis_name="s", num_cores=2, num_subcores=16)


def _deg_body(row2_hbm, degp_hbm, idx_v, ones_v, zseg_v, dacc):
    c = lax.axis_index("c")
    s = lax.axis_index("s")
    w = c * 16 + s

    for k in range(CH // 16):
        ones_v[pl.ds(k * 16, 16)] = jnp.ones((16,), jnp.float32)

    def zinit(i, _):
        zseg_v[pl.ds(i * 16, 16)] = jnp.zeros((16,), jnp.float32)
        return 0
    lax.fori_loop(0, 40, zinit, 0)
    pltpu.sync_copy(zseg_v, dacc.at[pl.ds(s * 640, 640)])

    pltpu.sync_copy(row2_hbm.at[pl.ds(w * NCHUNK, NCHUNK)], idx_v)

    plsc.subcore_barrier()

    def scat(j, _):
        pltpu.sync_copy(ones_v, dacc.at[idx_v.at[j]], add=True)
        return 0
    lax.fori_loop(0, NCHUNK, scat, 0)

    plsc.subcore_barrier()
    pltpu.sync_copy(dacc.at[pl.ds(s * 640, 640)],
                    degp_hbm.at[pl.ds(c * NP + s * 640, 640)])


_deg_call = functools.partial(
    pl.kernel, _deg_body, mesh=_mesh,
    out_type=jax.ShapeDtypeStruct((2 * NP,), jnp.float32),
    scratch_types=[
        pltpu.VMEM((NCHUNK, CH), jnp.int32),
        pltpu.VMEM((CH,), jnp.float32),
        pltpu.VMEM((640,), jnp.float32),
        pltpu.VMEM_SHARED((NP,), jnp.float32),
    ])()


def _prop_common(y_hbm, out_hbm, rows_v, sidx_v, buf_v, acc, sem0, sem1, c, s):
    def start2(j, slot, sem, t):
        pltpu.async_copy(
            y_hbm.at[rows_v.at[pl.ds(j * CH, CH)], pl.ds(t * DS, DS)],
            buf_v.at[slot], sem)

    def wait(slot, sem):
        pltpu.make_async_copy(y_hbm.at[pl.ds(0, CH), pl.ds(0, DS)],
                              buf_v.at[slot], sem).wait()

    for t in (0, 1):
        def zfill(i, _):
            buf_v[0, i // 8, pl.ds((i % 8) * 16, 16)] = jnp.zeros(
                (16,), jnp.float32)
            return 0
        lax.fori_loop(0, (CH * DS) // 16, zfill, 0)

        def zacc(i, _):
            pltpu.sync_copy(buf_v.at[0],
                            acc.at[pl.ds(s * 640 + i * CH, CH)])
            return 0
        lax.fori_loop(0, 640 // CH, zacc, 0)
        plsc.subcore_barrier()

        start2(0, 0, sem0, t)

        def body(i, _):
            j0 = 2 * i
            start2(j0 + 1, 1, sem1, t)
            wait(0, sem0)
            pltpu.sync_copy(buf_v.at[0], acc.at[sidx_v.at[j0]], add=True)

            @pl.when(i < NCH2 - 1)
            def _():
                start2(j0 + 2, 0, sem0, t)
            wait(1, sem1)
            pltpu.sync_copy(buf_v.at[1], acc.at[sidx_v.at[j0 + 1]], add=True)
            return 0
        lax.fori_loop(0, NCH2, body, 0)

        plsc.subcore_barrier()
        pltpu.sync_copy(
            acc.at[pl.ds(s * 640, 640)],
            out_hbm.at[pl.ds(c * NP + s * 640, 640), pl.ds(t * DS, DS)])
        plsc.subcore_barrier()


def _stage_edges(rowf_hbm, col2_hbm, rows_v, sidx_v, w):
    pltpu.sync_copy(rowf_hbm.at[pl.ds(w * NCHUNK * CH, NCHUNK * CH)], rows_v)
    pltpu.sync_copy(col2_hbm.at[pl.ds(w * NCHUNK, NCHUNK)], sidx_v)


def _prop_body(y_hbm, rowf_hbm, col2_hbm, out_hbm,
               rows_v, sidx_v, buf_v, acc, sem0, sem1):
    c = lax.axis_index("c")
    s = lax.axis_index("s")
    _stage_edges(rowf_hbm, col2_hbm, rows_v, sidx_v, c * 16 + s)
    _prop_common(y_hbm, out_hbm, rows_v, sidx_v, buf_v, acc, sem0, sem1, c, s)


_scratch = [
    pltpu.VMEM((NCHUNK * CH,), jnp.int32),
    pltpu.VMEM((NCHUNK, CH), jnp.int32),
    pltpu.VMEM((2, CH, DS), jnp.float32),
    pltpu.VMEM_SHARED((NP, DS), jnp.float32),
    pltpu.SemaphoreType.DMA,
    pltpu.SemaphoreType.DMA,
]

_prop_call = functools.partial(
    pl.kernel, _prop_body, mesh=_mesh,
    out_type=jax.ShapeDtypeStruct((2 * NP, D), jnp.float32),
    scratch_types=_scratch)()


def _tca_body(d0, d1, x, h0, h1, yx, yh0, yh1):
    dinv = lax.rsqrt(d0[...] + d1[...] + 2.0)
    yx[...] = dinv * x[...]
    yh0[...] = dinv * h0[...]
    yh1[...] = dinv * h1[...]


def _tca(d0, d1, x, h0, h1):
    col = pl.BlockSpec((RB, 1), lambda i: (i, 0))
    mat = pl.BlockSpec((RB, D), lambda i: (i, 0))
    return pl.pallas_call(
        _tca_body,
        grid=(NP // RB,),
        in_specs=[col, col, mat, mat, mat],
        out_specs=[mat, mat, mat],
        out_shape=[jax.ShapeDtypeStruct((NP, D), jnp.float32)] * 3,
    )(d0, d1, x, h0, h1)


_M0 = pl.BlockSpec((RB, D), lambda i: (i, 0))
_M1 = pl.BlockSpec((RB, D), lambda i: (i + NP // RB, 0))


def _tcb_body(sx0, sx1, sh0, sh1, yx, yh, hi, d0, d1, ws, bs, z, t1, q):
    dinv = lax.rsqrt(d0[...] + d1[...] + 2.0)
    ax = dinv * (sx0[...] + sx1[...] + 2.0 * yx[...])
    ah = dinv * (sh0[...] + sh1[...] + 2.0 * yh[...])
    axh = jnp.concatenate([ax, ah], axis=1)
    g = lax.dot_general(axh, ws[...], (((1,), (0,)), ((), ())),
                        precision=lax.Precision.DEFAULT,
                        preferred_element_type=jnp.float32) + bs[...]
    z[...] = jax.nn.sigmoid(g[:, :D])
    r = jax.nn.sigmoid(g[:, D:2 * D])
    t1[...] = g[:, 2 * D:]
    q[...] = dinv * (r * hi[...])


def _tcb(sx, sh, yx, yh, hi, d0, d1, ws, bs):
    col = pl.BlockSpec((RB, 1), lambda i: (i, 0))
    mat = pl.BlockSpec((RB, D), lambda i: (i, 0))
    return pl.pallas_call(
        _tcb_body,
        grid=(NP // RB,),
        in_specs=[_M0, _M1, _M0, _M1, mat, mat, mat, col, col,
                  pl.BlockSpec((2 * D, 3 * D), lambda i: (0, 0)),
                  pl.BlockSpec((1, 3 * D), lambda i: (0, 0))],
        out_specs=[mat, mat, mat],
        out_shape=[jax.ShapeDtypeStruct((NP, D), jnp.float32)] * 3,
    )(sx, sx, sh, sh, yx, yh, hi, d0, d1, ws, bs)


def _tcc_body(s20, s21, q, t1, z, hi, d0, d1, w5, b5, hn, yn):
    dinv = lax.rsqrt(d0[...] + d1[...] + 2.0)
    arh = dinv * (s20[...] + s21[...] + 2.0 * q[...])
    ht = jnp.tanh(t1[...] + b5[...] +
                  lax.dot_general(arh, w5[...], (((1,), (0,)), ((), ())),
                                  precision=lax.Precision.DEFAULT,
                                  preferred_element_type=jnp.float32))
    hv = z[...] * hi[...] + (1.0 - z[...]) * ht
    hn[...] = hv
    yn[...] = dinv * hv


def _tcc(s2, q, t1, z, hi, d0, d1, w5, b5):
    col = pl.BlockSpec((RB, 1), lambda i: (i, 0))
    mat = pl.BlockSpec((RB, D), lambda i: (i, 0))
    return pl.pallas_call(
        _tcc_body,
        grid=(NP // RB,),
        in_specs=[_M0, _M1, mat, mat, mat, mat, col, col,
                  pl.BlockSpec((D, D), lambda i: (0, 0)),
                  pl.BlockSpec((1, D), lambda i: (0, 0))],
        out_specs=[mat, mat],
        out_shape=[jax.ShapeDtypeStruct((NP, D), jnp.float32)] * 2,
    )(s2, s2, q, t1, z, hi, d0, d1, w5, b5)


def kernel(inp, edgidx, h, W, b):
    epad = N + (jnp.arange(E2 - E, dtype=jnp.int32) % (NP - N))
    rowf = jnp.concatenate([edgidx[0].astype(jnp.int32), epad])
    row2 = rowf.reshape(EROWS, CH)
    col2 = jnp.concatenate([edgidx[1].astype(jnp.int32), epad]).reshape(EROWS, CH)
    pad = jnp.zeros((NP - N, D), jnp.float32)
    inp_p = jnp.concatenate([inp, pad], axis=0)
    h0_p = jnp.concatenate([h[0], pad], axis=0)
    h1_p = jnp.concatenate([h[1], pad], axis=0)

    degp = _deg_call(row2)
    d0 = degp[:NP].reshape(NP, 1)
    d1 = degp[NP:].reshape(NP, 1)

    yx, yh0, yh1 = _tca(d0, d1, inp_p, h0_p, h1_p)

    hs = [None, None]
    yin = yx
    for i in range(2):
        hp = h0_p if i == 0 else h1_p
        yh = yh0 if i == 0 else yh1
        ws = jnp.concatenate([
            jnp.concatenate([W[i, 0], W[i, 1], W[i, 2]], axis=1),
            jnp.concatenate([W[i, 3], W[i, 4], jnp.zeros((D, D), jnp.float32)],
                            axis=1)], axis=0)
        bs = jnp.concatenate([b[i, 0] + b[i, 3], b[i, 1] + b[i, 4],
                              b[i, 2]])[None, :]
        sx = _prop_call(yin, rowf, col2)
        sh = _prop_call(yh, rowf, col2)
        z, t1, q = _tcb(sx, sh, yin, yh, hp, d0, d1, ws, bs)
        s2 = _prop_call(q, rowf, col2)
        hn, yn = _tcc(s2, q, t1, z, hp, d0, d1, W[i, 5], b[i, 5][None, :])
        hs[i] = hn[:N]
        yin = yn

    h_out = jnp.stack(hs, axis=0)
    return (h_out, h_out)
